# baseline (speedup 1.0000x reference)
import functools

import numpy as np
import jax
import jax.numpy as jnp
from jax import lax
from jax.experimental import pallas as pl
from jax.experimental.pallas import tpu as pltpu
from jax.experimental.pallas import tpu_sc as plsc

NC, NS, L = 2, 16, 16
NW = NC * NS
W = 64


@functools.lru_cache(maxsize=None)
def _sc_scores(B, K, D):
    P = K + 1
    b_w = B // NW
    CB = 64
    NCH = b_w // CB
    PR = CB * P
    NIDX = PR // W
    NV = D // L
    CROWS = b_w // W
    XROWS = NCH * NIDX
    HROW = 8 * (P + 1) + L

    mesh = plsc.VectorSubcoreMesh(
        core_axis_name="c", subcore_axis_name="s", num_cores=NC, num_subcores=NS
    )

    @functools.partial(
        pl.kernel,
        out_type=jax.ShapeDtypeStruct((B, P * L), jnp.float32),
        mesh=mesh,
        scratch_types=[
            pltpu.VMEM((CROWS, W), jnp.int32),
            pltpu.VMEM((XROWS, W), jnp.int32),
            pltpu.VMEM((b_w // 8, HROW), jnp.float32),
            pltpu.VMEM((CB, 2 * D), jnp.float32),
            pltpu.VMEM((PR, 2 * D), jnp.float32),
            pltpu.VMEM((CB // 2, P * L), jnp.float32),
            pltpu.SemaphoreType.DMA,
            pltpu.SemaphoreType.DMA,
        ],
    )
    def k(cq_hbm, xq_hbm, hp_hbm, tw_hbm, cw_hbm, out_hbm,
          cqv, xqv, hpv, vbuf, ubuf, part, sem_v, sem_u):
        wid = lax.axis_index("s") * NC + lax.axis_index("c")
        pltpu.sync_copy(cq_hbm.at[pl.ds(wid * CROWS, CROWS)], cqv)
        pltpu.sync_copy(xq_hbm.at[pl.ds(wid * XROWS, XROWS)], xqv)
        pltpu.sync_copy(hp_hbm.at[pl.ds(wid * (b_w // 8), b_w // 8)], hpv)

        def chunk(c, carry):
            cp_v = pltpu.async_copy(tw_hbm.at[cqv.at[c]], vbuf, sem_v)
            cps = [
                pltpu.async_copy(cw_hbm.at[xqv.at[c * NIDX + t]],
                                 ubuf.at[pl.ds(t * W, W)], sem_u)
                for t in range(NIDX)
            ]
            cp_v.wait()
            for cp in cps:
                cp.wait()

            def half(hb, carry3):
                def body(b2, carry2):
                    b = hb * (CB // 2) + b2
                    bw = c * CB + b
                    hvec = hpv[bw // 8, pl.ds((bw % 8) * (P + 1), L)]
                    chf = hvec[P]
                    vr = []
                    for i in range(NV):
                        vlo = vbuf[b, pl.ds(i * L, L)]
                        vhi = vbuf[b, pl.ds(D + i * L, L)]
                        vr.append(vlo + (vhi - vlo) * chf)
                    for j in range(P):
                        p = b * P + j
                        slo = vr[0] * ubuf[p, pl.ds(0, L)]
                        shi = vr[0] * ubuf[p, pl.ds(D, L)]
                        for i in range(1, NV):
                            slo = slo + vr[i] * ubuf[p, pl.ds(i * L, L)]
                            shi = shi + vr[i] * ubuf[p, pl.ds(D + i * L, L)]
                        part[b2, pl.ds(j * L, L)] = slo + (shi - slo) * hvec[j]
                    return carry2

                lax.fori_loop(0, CB // 2, body, 0)
                pltpu.sync_copy(
                    part,
                    out_hbm.at[pl.ds(wid * b_w + c * CB + hb * (CB // 2),
                                     CB // 2)],
                )
                return carry3

            lax.fori_loop(0, 2, half, 0)
            return carry

        lax.fori_loop(0, NCH, chunk, 0)

    return k


@functools.lru_cache(maxsize=None)
def _tc_loss(B, P):
    BLK = 512
    G = B // BLK

    def body(x_ref, m_ref, out_ref):
        x = x_ref[...]
        s = jnp.dot(x, m_ref[...], preferred_element_type=jnp.float32)
        col = lax.broadcasted_iota(jnp.int32, s.shape, 1)
        t = jnp.where(col == 0, s, -s)
        loss = -jnp.log(jax.nn.sigmoid(t) + 1e-09)

        @pl.when(pl.program_id(0) == 0)
        def _():
            out_ref[...] = jnp.zeros((1, 1), jnp.float32)

        out_ref[...] = out_ref[...] + jnp.sum(loss)

        @pl.when(pl.program_id(0) == G - 1)
        def _():
            out_ref[...] = out_ref[...] / B

    return pl.pallas_call(
        body,
        grid=(G,),
        in_specs=[
            pl.BlockSpec((BLK, P * L), lambda i: (i, 0)),
            pl.BlockSpec((P * L, P), lambda i: (0, 0)),
        ],
        out_specs=pl.BlockSpec((1, 1), lambda i: (0, 0)),
        out_shape=jax.ShapeDtypeStruct((1, 1), jnp.float32),
    )


@functools.lru_cache(maxsize=None)
def _lane_sum_matrix(P):
    m = np.zeros((P * L, P), dtype=np.float32)
    for j in range(P):
        m[j * L:(j + 1) * L, j] = 1.0
    return jnp.asarray(m)


def kernel(center_ids, pos_ctx_ids, neg_ctx_ids, target_W, context_W):
    B = center_ids.shape[0]
    K = neg_ctx_ids.shape[1]
    V, D = target_W.shape
    P = K + 1
    hrow = 8 * (P + 1) + L
    cen = center_ids.astype(jnp.int32)
    ctx = jnp.concatenate(
        [pos_ctx_ids.astype(jnp.int32)[:, None], neg_ctx_ids.astype(jnp.int32)],
        axis=1,
    ).reshape(B * P)
    cq = (cen // 2).reshape(B // W, W)
    xq = (ctx // 2).reshape(B * P // W, W)
    hp = jnp.concatenate(
        [(ctx % 2).astype(jnp.float32).reshape(B, P),
         (cen % 2).astype(jnp.float32)[:, None]],
        axis=1,
    ).reshape(B // 8, 8 * (P + 1))
    hp = jnp.pad(hp, ((0, 0), (0, hrow - 8 * (P + 1))))
    tw2 = target_W.reshape(V // 2, 2 * D)
    cw2 = context_W.reshape(V // 2, 2 * D)
    part = _sc_scores(B, K, D)(cq, xq, hp, tw2, cw2)
    out = _tc_loss(B, P)(part, _lane_sum_matrix(P))
    return out[0, 0]
